# Initial kernel scaffold; baseline (speedup 1.0000x reference)
import jax
import jax.numpy as jnp
from jax.experimental import pallas as pl


def kernel(sentences1, sentences2, table):
    raise NotImplementedError("write your pallas kernel here")


# speedup vs baseline: 1.4000x; 1.4000x over previous
import functools

import jax
import jax.numpy as jnp
from jax import lax
from jax.experimental import pallas as pl
from jax.experimental.pallas import tpu as pltpu
from jax.experimental.pallas import tpu_sc as plsc

B = 4096
L = 50
EMB = 64
N = 2 * B

_info = plsc.get_sparse_core_info()
NC, NS = _info.num_cores, _info.num_subcores
NW = NC * NS
SENT_PER_W = N // NW
C = 8
CROWS = C * L
NCHUNK = SENT_PER_W // C

_sc_mesh = plsc.VectorSubcoreMesh(core_axis_name="c", subcore_axis_name="s")


@functools.partial(
    pl.kernel,
    mesh=_sc_mesh,
    out_type=jax.ShapeDtypeStruct((N, EMB), jnp.float32),
    scratch_types=[
        pltpu.VMEM((CROWS,), jnp.int32),
        pltpu.VMEM((CROWS, EMB), jnp.float32),
        pltpu.VMEM((C, EMB), jnp.float32),
        pltpu.SemaphoreType.DMA,
    ],
    compiler_params=pltpu.CompilerParams(use_tc_tiling_on_sc=False),
)
def _pooled_gather(idx_hbm, table_hbm, out_hbm, idx_v, rows_v, out_v, sem):
    wid = lax.axis_index("s") * NC + lax.axis_index("c")
    inv = jnp.float32(1.0 / L)

    def chunk_body(c, carry):
        base_s = wid * SENT_PER_W + c * C
        pltpu.sync_copy(idx_hbm.at[pl.ds(base_s * L, CROWS)], idx_v)
        pltpu.async_copy(table_hbm.at[idx_v], rows_v, sem).wait()
        for s in range(C):
            def row_body(r, accs, _s=s):
                base = _s * L + r
                return tuple(
                    accs[jj] + rows_v[base, pl.ds(jj * 16, 16)] for jj in range(4)
                )

            accs = lax.fori_loop(
                0, L, row_body,
                tuple(jnp.zeros((16,), jnp.float32) for _ in range(4)),
            )
            for jj in range(4):
                out_v[s, pl.ds(jj * 16, 16)] = accs[jj] * inv
        pltpu.sync_copy(out_v, out_hbm.at[pl.ds(base_s, C)])
        return carry

    lax.fori_loop(0, NCHUNK, chunk_body, 0)


BM = 512
BN = 512


def _loss_body(s1_ref, s2_ref, out_ref):
    i = pl.program_id(0)
    j = pl.program_id(1)
    logits = lax.dot_general(
        s1_ref[...], s2_ref[...],
        dimension_numbers=(((1,), (1,)), ((), ())),
        preferred_element_type=jnp.float32,
    )
    loss = jnp.maximum(logits, 0.0) + jnp.log1p(jnp.exp(-jnp.abs(logits)))
    row = lax.broadcasted_iota(jnp.int32, (BM, BN), 0)
    col = lax.broadcasted_iota(jnp.int32, (BM, BN), 1)
    diag = jnp.where((row == col) & (i == j), logits, 0.0)
    contrib = jnp.sum(loss - diag)

    @pl.when((i == 0) & (j == 0))
    def _():
        out_ref[0, 0] = 0.0

    out_ref[0, 0] += contrib


_loss_call = pl.pallas_call(
    _loss_body,
    grid=(B // BM, B // BN),
    in_specs=[
        pl.BlockSpec((BM, EMB), lambda i, j: (i, 0)),
        pl.BlockSpec((BN, EMB), lambda i, j: (j, 0)),
    ],
    out_specs=pl.BlockSpec((1, 1), lambda i, j: (0, 0), memory_space=pltpu.SMEM),
    out_shape=jax.ShapeDtypeStruct((1, 1), jnp.float32),
)


def kernel(sentences1, sentences2, table):
    idx = jnp.concatenate([sentences1, sentences2], axis=0).reshape(-1)
    pooled = _pooled_gather(idx, table)
    total = _loss_call(pooled[:B], pooled[B:])
    return total[0, 0] / jnp.float32(B * B)


# speedup vs baseline: 1.6818x; 1.2013x over previous
import functools

import jax
import jax.numpy as jnp
from jax import lax
from jax.experimental import pallas as pl
from jax.experimental.pallas import tpu as pltpu
from jax.experimental.pallas import tpu_sc as plsc

B = 4096
L = 50
EMB = 64
N = 2 * B
V = 1000000

_info = plsc.get_sparse_core_info()
NC, NS = _info.num_cores, _info.num_subcores
NW = NC * NS
SENT_PER_W = N // NW
C = 8
CROWS = C * L
NCHUNK = SENT_PER_W // C

PW = 2048


def _pack_body(tt_ref, out_ref):
    t = jnp.transpose(tt_ref[...])
    wide = jnp.concatenate([t, t], axis=1)
    out_ref[...] = wide.reshape(PW * 128)


_pack_call = pl.pallas_call(
    _pack_body,
    grid=(pl.cdiv(V, PW),),
    in_specs=[pl.BlockSpec((EMB, PW), lambda w: (0, w))],
    out_specs=pl.BlockSpec((PW * 128,), lambda w: (w,)),
    out_shape=jax.ShapeDtypeStruct((V * 128,), jnp.float32),
)

_sc_mesh = plsc.VectorSubcoreMesh(core_axis_name="c", subcore_axis_name="s")


@functools.partial(
    pl.kernel,
    mesh=_sc_mesh,
    out_type=jax.ShapeDtypeStruct((N, EMB), jnp.float32),
    scratch_types=[
        pltpu.VMEM((SENT_PER_W * L,), jnp.int32),
        pltpu.VMEM((CROWS, EMB), jnp.float32),
        pltpu.VMEM((CROWS, EMB), jnp.float32),
        pltpu.VMEM((C, EMB), jnp.float32),
        pltpu.SemaphoreType.DMA,
        pltpu.SemaphoreType.DMA,
    ],
    compiler_params=pltpu.CompilerParams(use_tc_tiling_on_sc=False),
)
def _pooled_gather(idx_hbm, table_hbm, out_hbm, idx_all, rows0, rows1, out_v, sem0, sem1):
    wid = lax.axis_index("s") * NC + lax.axis_index("c")
    inv = jnp.float32(1.0 / L)
    rowsb = (rows0, rows1)
    semb = (sem0, sem1)

    pltpu.sync_copy(idx_hbm.at[pl.ds(wid * (SENT_PER_W * L), SENT_PER_W * L)], idx_all)

    def start_gather(c, b):
        pltpu.async_copy(
            table_hbm.at[idx_all.at[pl.ds(c * CROWS, CROWS)]], rowsb[b], semb[b]
        )

    start_gather(0, 0)

    def outer(g, carry):
        for b in range(2):
            c = g * 2 + b

            @pl.when(c + 1 < NCHUNK)
            def _():
                start_gather(c + 1, 1 - b)

            pltpu.make_async_copy(
                table_hbm.at[idx_all.at[pl.ds(0, CROWS)]], rowsb[b], semb[b]
            ).wait()
            rows_v = rowsb[b]
            for s in range(C):
                def row_body(r, accs, _s=s):
                    base = _s * L + r
                    return tuple(
                        accs[jj] + rows_v[base, pl.ds(jj * 16, 16)] for jj in range(4)
                    )

                accs = lax.fori_loop(
                    0, L, row_body,
                    tuple(jnp.zeros((16,), jnp.float32) for _ in range(4)),
                    unroll=10,
                )
                for jj in range(4):
                    out_v[s, pl.ds(jj * 16, 16)] = accs[jj] * inv
            base_s = wid * SENT_PER_W + c * C
            pltpu.sync_copy(out_v, out_hbm.at[pl.ds(base_s, C)])
        return carry

    lax.fori_loop(0, NCHUNK // 2, outer, 0)


BM = 512
BN = 512


def _loss_body(s1_ref, s2_ref, out_ref):
    i = pl.program_id(0)
    j = pl.program_id(1)
    logits = lax.dot_general(
        s1_ref[...], s2_ref[...],
        dimension_numbers=(((1,), (1,)), ((), ())),
        preferred_element_type=jnp.float32,
    )
    loss = jnp.maximum(logits, 0.0) + jnp.log1p(jnp.exp(-jnp.abs(logits)))
    contrib = jnp.sum(loss)

    @pl.when((i == 0) & (j == 0))
    def _():
        out_ref[0, 0] = 0.0

    out_ref[0, 0] += contrib

    @pl.when(i == j)
    def _():
        row = lax.broadcasted_iota(jnp.int32, (BM, BN), 0)
        col = lax.broadcasted_iota(jnp.int32, (BM, BN), 1)
        diag = jnp.sum(jnp.where(row == col, logits, 0.0))
        out_ref[0, 0] += -diag


_loss_call = pl.pallas_call(
    _loss_body,
    grid=(B // BM, B // BN),
    in_specs=[
        pl.BlockSpec((BM, EMB), lambda i, j: (i, 0)),
        pl.BlockSpec((BN, EMB), lambda i, j: (j + B // BN, 0)),
    ],
    out_specs=pl.BlockSpec((1, 1), lambda i, j: (0, 0), memory_space=pltpu.SMEM),
    out_shape=jax.ShapeDtypeStruct((1, 1), jnp.float32),
)


def kernel(sentences1, sentences2, table):
    idx2 = jnp.concatenate([sentences1, sentences2], axis=0).reshape(-1) * 2
    packed = _pack_call(table.T)
    table_lin = packed.reshape(2 * V, EMB)
    pooled = _pooled_gather(idx2, table_lin)
    total = _loss_call(pooled, pooled)
    return total[0, 0] / jnp.float32(B * B)


# speedup vs baseline: 2.2672x; 1.3481x over previous
import functools

import jax
import jax.numpy as jnp
from jax import lax
from jax.experimental import pallas as pl
from jax.experimental.pallas import tpu as pltpu
from jax.experimental.pallas import tpu_sc as plsc

B = 4096
L = 50
EMB = 64
N = 2 * B
V = 1000000

_info = plsc.get_sparse_core_info()
NC, NS = _info.num_cores, _info.num_subcores
NW = NC * NS
SENT_PER_W = N // NW
C = 8
CROWS = C * L
NCHUNK = SENT_PER_W // C

PW = 4096
NB = pl.cdiv(V, PW)
HALF = PW // 2


def _pack_body(tt_ref, out_ref):
    t = jnp.transpose(tt_ref[...])
    wide = jnp.concatenate([t[:HALF], t[HALF:]], axis=1)
    out_ref[...] = wide.reshape(HALF * 128)


_pack_call = pl.pallas_call(
    _pack_body,
    grid=(NB,),
    in_specs=[pl.BlockSpec((EMB, PW), lambda w: (0, w))],
    out_specs=pl.BlockSpec((HALF * 128,), lambda w: (w,)),
    out_shape=jax.ShapeDtypeStruct((NB * HALF * 128,), jnp.float32),
)

_sc_mesh = plsc.VectorSubcoreMesh(core_axis_name="c", subcore_axis_name="s")


@functools.partial(
    pl.kernel,
    mesh=_sc_mesh,
    out_type=jax.ShapeDtypeStruct((N, EMB), jnp.float32),
    scratch_types=[
        pltpu.VMEM((SENT_PER_W * L,), jnp.int32),
        pltpu.VMEM((CROWS, EMB), jnp.float32),
        pltpu.VMEM((CROWS, EMB), jnp.float32),
        pltpu.VMEM((C, EMB), jnp.float32),
        pltpu.SemaphoreType.DMA,
        pltpu.SemaphoreType.DMA,
    ],
    compiler_params=pltpu.CompilerParams(use_tc_tiling_on_sc=False),
)
def _pooled_gather(idx_hbm, table_hbm, out_hbm, idx_all, rows0, rows1, out_v, sem0, sem1):
    wid = lax.axis_index("s") * NC + lax.axis_index("c")
    inv = jnp.float32(1.0 / L)
    rowsb = (rows0, rows1)
    semb = (sem0, sem1)

    pltpu.sync_copy(idx_hbm.at[pl.ds(wid * (SENT_PER_W * L), SENT_PER_W * L)], idx_all)

    def start_gather(c, b):
        pltpu.async_copy(
            table_hbm.at[idx_all.at[pl.ds(c * CROWS, CROWS)]], rowsb[b], semb[b]
        )

    start_gather(0, 0)

    def outer(g, carry):
        for b in range(2):
            c = g * 2 + b

            @pl.when(c + 1 < NCHUNK)
            def _():
                start_gather(c + 1, 1 - b)

            pltpu.make_async_copy(
                table_hbm.at[idx_all.at[pl.ds(0, CROWS)]], rowsb[b], semb[b]
            ).wait()
            rows_v = rowsb[b]
            for s in range(C):
                def row_body(r, accs, _s=s):
                    base = _s * L + r
                    return tuple(
                        accs[jj] + rows_v[base, pl.ds(jj * 16, 16)] for jj in range(4)
                    )

                accs = lax.fori_loop(
                    0, L, row_body,
                    tuple(jnp.zeros((16,), jnp.float32) for _ in range(4)),
                    unroll=10,
                )
                for jj in range(4):
                    out_v[s, pl.ds(jj * 16, 16)] = accs[jj] * inv
            base_s = wid * SENT_PER_W + c * C
            pltpu.sync_copy(out_v, out_hbm.at[pl.ds(base_s, C)])
        return carry

    lax.fori_loop(0, NCHUNK // 2, outer, 0)


BM = 512
BN = 512


def _loss_body(s1_ref, s2_ref, out_ref):
    i = pl.program_id(0)
    j = pl.program_id(1)
    logits = lax.dot_general(
        s1_ref[...], s2_ref[...],
        dimension_numbers=(((1,), (1,)), ((), ())),
        preferred_element_type=jnp.float32,
    )
    loss = jnp.maximum(logits, 0.0) + jnp.log1p(jnp.exp(-jnp.abs(logits)))
    contrib = jnp.sum(loss)

    @pl.when((i == 0) & (j == 0))
    def _():
        out_ref[0, 0] = 0.0

    out_ref[0, 0] += contrib

    @pl.when(i == j)
    def _():
        row = lax.broadcasted_iota(jnp.int32, (BM, BN), 0)
        col = lax.broadcasted_iota(jnp.int32, (BM, BN), 1)
        diag = jnp.sum(jnp.where(row == col, logits, 0.0))
        out_ref[0, 0] += -diag


_loss_call = pl.pallas_call(
    _loss_body,
    grid=(B // BM, B // BN),
    in_specs=[
        pl.BlockSpec((BM, EMB), lambda i, j: (i, 0)),
        pl.BlockSpec((BN, EMB), lambda i, j: (j + B // BN, 0)),
    ],
    out_specs=pl.BlockSpec((1, 1), lambda i, j: (0, 0), memory_space=pltpu.SMEM),
    out_shape=jax.ShapeDtypeStruct((1, 1), jnp.float32),
)


def kernel(sentences1, sentences2, table):
    idx = jnp.concatenate([sentences1, sentences2], axis=0).reshape(-1)
    k = idx & (PW - 1)
    fidx = (idx & ~(PW - 1)) + jnp.where(k < HALF, k << 1, (k << 1) - (PW - 1))
    packed = _pack_call(table.T)
    table_lin = packed.reshape(NB * PW, EMB)
    pooled = _pooled_gather(fidx, table_lin)
    total = _loss_call(pooled, pooled)
    return total[0, 0] / jnp.float32(B * B)


# speedup vs baseline: 2.6245x; 1.1576x over previous
import functools

import jax
import jax.numpy as jnp
from jax import lax
from jax.experimental import pallas as pl
from jax.experimental.pallas import tpu as pltpu
from jax.experimental.pallas import tpu_sc as plsc

B = 4096
L = 50
EMB = 64
N = 2 * B
V = 1000000

_info = plsc.get_sparse_core_info()
NC, NS = _info.num_cores, _info.num_subcores
NW = NC * NS
SENT_PER_W = N // NW
C = 8
CROWS = C * L
NCHUNK = SENT_PER_W // C

PW = 8192
NB = pl.cdiv(V, PW)
HALF = PW // 2


def _pack_body(tt_ref, out_ref):
    t = jnp.transpose(tt_ref[...])
    wide = jnp.concatenate([t[:HALF], t[HALF:]], axis=1)
    out_ref[...] = wide.reshape(HALF * 128)


_pack_call = pl.pallas_call(
    _pack_body,
    grid=(NB,),
    in_specs=[pl.BlockSpec((EMB, PW), lambda w: (0, w))],
    out_specs=pl.BlockSpec((HALF * 128,), lambda w: (w,)),
    out_shape=jax.ShapeDtypeStruct((NB * HALF * 128,), jnp.float32),
)

_sc_mesh = plsc.VectorSubcoreMesh(core_axis_name="c", subcore_axis_name="s")


@functools.partial(
    pl.kernel,
    mesh=_sc_mesh,
    out_type=jax.ShapeDtypeStruct((N, EMB), jnp.float32),
    scratch_types=[
        pltpu.VMEM((SENT_PER_W * L,), jnp.int32),
        pltpu.VMEM((CROWS, EMB), jnp.float32),
        pltpu.VMEM((CROWS, EMB), jnp.float32),
        pltpu.VMEM((C, EMB), jnp.float32),
        pltpu.SemaphoreType.DMA,
        pltpu.SemaphoreType.DMA,
    ],
    compiler_params=pltpu.CompilerParams(use_tc_tiling_on_sc=False),
)
def _pooled_gather(idx_hbm, table_hbm, out_hbm, idx_all, rows0, rows1, out_v, sem0, sem1):
    wid = lax.axis_index("s") * NC + lax.axis_index("c")
    inv = jnp.float32(1.0 / L)
    rowsb = (rows0, rows1)
    semb = (sem0, sem1)

    pltpu.sync_copy(idx_hbm.at[pl.ds(wid * (SENT_PER_W * L), SENT_PER_W * L)], idx_all)

    def start_gather(c, b):
        pltpu.async_copy(
            table_hbm.at[idx_all.at[pl.ds(c * CROWS, CROWS)]], rowsb[b], semb[b]
        )

    start_gather(0, 0)

    def outer(g, carry):
        for b in range(2):
            c = g * 2 + b

            @pl.when(c + 1 < NCHUNK)
            def _():
                start_gather(c + 1, 1 - b)

            pltpu.make_async_copy(
                table_hbm.at[idx_all.at[pl.ds(0, CROWS)]], rowsb[b], semb[b]
            ).wait()
            rows_v = rowsb[b]
            for s in range(C):
                def row_body(r, accs, _s=s):
                    base = _s * L + r
                    return tuple(
                        accs[jj] + rows_v[base, pl.ds(jj * 16, 16)] for jj in range(4)
                    )

                accs = lax.fori_loop(
                    0, L, row_body,
                    tuple(jnp.zeros((16,), jnp.float32) for _ in range(4)),
                    unroll=10,
                )
                for jj in range(4):
                    out_v[s, pl.ds(jj * 16, 16)] = accs[jj] * inv
            base_s = wid * SENT_PER_W + c * C
            pltpu.sync_copy(out_v, out_hbm.at[pl.ds(base_s, C)])
        return carry

    lax.fori_loop(0, NCHUNK // 2, outer, 0)


BM = 512
BN = 512


def _loss_body(s1_ref, s2_ref, out_ref):
    i = pl.program_id(0)
    j = pl.program_id(1)
    logits = lax.dot_general(
        s1_ref[...], s2_ref[...],
        dimension_numbers=(((1,), (1,)), ((), ())),
        preferred_element_type=jnp.float32,
    )
    loss = jnp.maximum(logits, 0.0) + jnp.log1p(jnp.exp(-jnp.abs(logits)))
    contrib = jnp.sum(loss)

    @pl.when((i == 0) & (j == 0))
    def _():
        out_ref[0, 0] = 0.0

    out_ref[0, 0] += contrib

    @pl.when(i == j)
    def _():
        row = lax.broadcasted_iota(jnp.int32, (BM, BN), 0)
        col = lax.broadcasted_iota(jnp.int32, (BM, BN), 1)
        diag = jnp.sum(jnp.where(row == col, logits, 0.0))
        out_ref[0, 0] += -diag


_loss_call = pl.pallas_call(
    _loss_body,
    grid=(B // BM, B // BN),
    in_specs=[
        pl.BlockSpec((BM, EMB), lambda i, j: (i, 0)),
        pl.BlockSpec((BN, EMB), lambda i, j: (j + B // BN, 0)),
    ],
    out_specs=pl.BlockSpec((1, 1), lambda i, j: (0, 0), memory_space=pltpu.SMEM),
    out_shape=jax.ShapeDtypeStruct((1, 1), jnp.float32),
)


def kernel(sentences1, sentences2, table):
    idx = jnp.concatenate([sentences1, sentences2], axis=0).reshape(-1)
    k = idx & (PW - 1)
    fidx = (idx & ~(PW - 1)) + jnp.where(k < HALF, k << 1, (k << 1) - (PW - 1))
    packed = _pack_call(table.T)
    table_lin = packed.reshape(NB * PW, EMB)
    pooled = _pooled_gather(fidx, table_lin)
    total = _loss_call(pooled, pooled)
    return total[0, 0] / jnp.float32(B * B)


# speedup vs baseline: 2.8329x; 1.0794x over previous
import functools

import jax
import jax.numpy as jnp
from jax import lax
from jax.experimental import pallas as pl
from jax.experimental.pallas import tpu as pltpu
from jax.experimental.pallas import tpu_sc as plsc

B = 4096
L = 50
EMB = 64
N = 2 * B
V = 1000000

_info = plsc.get_sparse_core_info()
NC, NS = _info.num_cores, _info.num_subcores
NW = NC * NS
SENT_PER_W = N // NW
C = 8
CROWS = C * L
NCHUNK = SENT_PER_W // C

PW = 16384
NB = pl.cdiv(V, PW)
HALF = PW // 2


def _pack_body(tt_ref, out_ref):
    t = jnp.transpose(tt_ref[...])
    wide = jnp.concatenate([t[:HALF], t[HALF:]], axis=1)
    out_ref[...] = wide.reshape(HALF * 128)


_pack_call = pl.pallas_call(
    _pack_body,
    grid=(NB,),
    in_specs=[pl.BlockSpec((EMB, PW), lambda w: (0, w))],
    out_specs=pl.BlockSpec((HALF * 128,), lambda w: (w,)),
    out_shape=jax.ShapeDtypeStruct((NB * HALF * 128,), jnp.float32),
)

_sc_mesh = plsc.VectorSubcoreMesh(core_axis_name="c", subcore_axis_name="s")


@functools.partial(
    pl.kernel,
    mesh=_sc_mesh,
    out_type=jax.ShapeDtypeStruct((N, EMB), jnp.float32),
    scratch_types=[
        pltpu.VMEM((SENT_PER_W * L,), jnp.int32),
        pltpu.VMEM((CROWS, EMB), jnp.float32),
        pltpu.VMEM((CROWS, EMB), jnp.float32),
        pltpu.VMEM((C, EMB), jnp.float32),
        pltpu.SemaphoreType.DMA,
        pltpu.SemaphoreType.DMA,
    ],
    compiler_params=pltpu.CompilerParams(use_tc_tiling_on_sc=False),
)
def _pooled_gather(idx_hbm, table_hbm, out_hbm, idx_all, rows0, rows1, out_v, sem0, sem1):
    wid = lax.axis_index("s") * NC + lax.axis_index("c")
    inv = jnp.float32(1.0 / L)
    rowsb = (rows0, rows1)
    semb = (sem0, sem1)

    pltpu.sync_copy(idx_hbm.at[pl.ds(wid * (SENT_PER_W * L), SENT_PER_W * L)], idx_all)

    def start_gather(c, b):
        pltpu.async_copy(
            table_hbm.at[idx_all.at[pl.ds(c * CROWS, CROWS)]], rowsb[b], semb[b]
        )

    start_gather(0, 0)

    def outer(g, carry):
        for b in range(2):
            c = g * 2 + b

            @pl.when(c + 1 < NCHUNK)
            def _():
                start_gather(c + 1, 1 - b)

            pltpu.make_async_copy(
                table_hbm.at[idx_all.at[pl.ds(0, CROWS)]], rowsb[b], semb[b]
            ).wait()
            rows_v = rowsb[b]
            for s in range(C):
                def row_body(r, accs, _s=s):
                    base = _s * L + r
                    return tuple(
                        accs[jj] + rows_v[base, pl.ds(jj * 16, 16)] for jj in range(4)
                    )

                accs = lax.fori_loop(
                    0, L, row_body,
                    tuple(jnp.zeros((16,), jnp.float32) for _ in range(4)),
                    unroll=10,
                )
                for jj in range(4):
                    out_v[s, pl.ds(jj * 16, 16)] = accs[jj] * inv
            base_s = wid * SENT_PER_W + c * C
            pltpu.sync_copy(out_v, out_hbm.at[pl.ds(base_s, C)])
        return carry

    lax.fori_loop(0, NCHUNK // 2, outer, 0)


BM = 512
BN = 512


def _loss_body(s1_ref, s2_ref, out_ref):
    i = pl.program_id(0)
    j = pl.program_id(1)
    logits = lax.dot_general(
        s1_ref[...], s2_ref[...],
        dimension_numbers=(((1,), (1,)), ((), ())),
        preferred_element_type=jnp.float32,
    )
    loss = jnp.maximum(logits, 0.0) + jnp.log1p(jnp.exp(-jnp.abs(logits)))
    contrib = jnp.sum(loss)

    @pl.when((i == 0) & (j == 0))
    def _():
        out_ref[0, 0] = 0.0

    out_ref[0, 0] += contrib

    @pl.when(i == j)
    def _():
        row = lax.broadcasted_iota(jnp.int32, (BM, BN), 0)
        col = lax.broadcasted_iota(jnp.int32, (BM, BN), 1)
        diag = jnp.sum(jnp.where(row == col, logits, 0.0))
        out_ref[0, 0] += -diag


_loss_call = pl.pallas_call(
    _loss_body,
    grid=(B // BM, B // BN),
    in_specs=[
        pl.BlockSpec((BM, EMB), lambda i, j: (i, 0)),
        pl.BlockSpec((BN, EMB), lambda i, j: (j + B // BN, 0)),
    ],
    out_specs=pl.BlockSpec((1, 1), lambda i, j: (0, 0), memory_space=pltpu.SMEM),
    out_shape=jax.ShapeDtypeStruct((1, 1), jnp.float32),
)


def kernel(sentences1, sentences2, table):
    idx = jnp.concatenate([sentences1, sentences2], axis=0).reshape(-1)
    k = idx & (PW - 1)
    fidx = (idx & ~(PW - 1)) + jnp.where(k < HALF, k << 1, (k << 1) - (PW - 1))
    packed = _pack_call(table.T)
    table_lin = packed.reshape(NB * PW, EMB)
    pooled = _pooled_gather(fidx, table_lin)
    total = _loss_call(pooled, pooled)
    return total[0, 0] / jnp.float32(B * B)


# speedup vs baseline: 2.9429x; 1.0388x over previous
import functools

import jax
import jax.numpy as jnp
from jax import lax
from jax.experimental import pallas as pl
from jax.experimental.pallas import tpu as pltpu
from jax.experimental.pallas import tpu_sc as plsc

B = 4096
L = 50
EMB = 64
N = 2 * B
V = 1000000

_info = plsc.get_sparse_core_info()
NC, NS = _info.num_cores, _info.num_subcores
NW = NC * NS
SENT_PER_W = N // NW
C = 8
CROWS = C * L
NCHUNK = SENT_PER_W // C

PW = 32768
NB = pl.cdiv(V, PW)
HALF = PW // 2


def _pack_body(tt_ref, out_ref):
    t = jnp.transpose(tt_ref[...])
    wide = jnp.concatenate([t[:HALF], t[HALF:]], axis=1)
    out_ref[...] = wide.reshape(HALF * 128)


_pack_call = pl.pallas_call(
    _pack_body,
    grid=(NB,),
    in_specs=[pl.BlockSpec((EMB, PW), lambda w: (0, w))],
    out_specs=pl.BlockSpec((HALF * 128,), lambda w: (w,)),
    out_shape=jax.ShapeDtypeStruct((NB * HALF * 128,), jnp.float32),
)

_sc_mesh = plsc.VectorSubcoreMesh(core_axis_name="c", subcore_axis_name="s")


@functools.partial(
    pl.kernel,
    mesh=_sc_mesh,
    out_type=jax.ShapeDtypeStruct((N, EMB), jnp.float32),
    scratch_types=[
        pltpu.VMEM((SENT_PER_W * L,), jnp.int32),
        pltpu.VMEM((CROWS, EMB), jnp.float32),
        pltpu.VMEM((CROWS, EMB), jnp.float32),
        pltpu.VMEM((C, EMB), jnp.float32),
        pltpu.SemaphoreType.DMA,
        pltpu.SemaphoreType.DMA,
    ],
    compiler_params=pltpu.CompilerParams(use_tc_tiling_on_sc=False),
)
def _pooled_gather(idx_hbm, table_hbm, out_hbm, idx_all, rows0, rows1, out_v, sem0, sem1):
    wid = lax.axis_index("s") * NC + lax.axis_index("c")
    inv = jnp.float32(1.0 / L)
    rowsb = (rows0, rows1)
    semb = (sem0, sem1)

    pltpu.sync_copy(idx_hbm.at[pl.ds(wid * (SENT_PER_W * L), SENT_PER_W * L)], idx_all)

    def start_gather(c, b):
        pltpu.async_copy(
            table_hbm.at[idx_all.at[pl.ds(c * CROWS, CROWS)]], rowsb[b], semb[b]
        )

    start_gather(0, 0)

    def outer(g, carry):
        for b in range(2):
            c = g * 2 + b

            @pl.when(c + 1 < NCHUNK)
            def _():
                start_gather(c + 1, 1 - b)

            pltpu.make_async_copy(
                table_hbm.at[idx_all.at[pl.ds(0, CROWS)]], rowsb[b], semb[b]
            ).wait()
            rows_v = rowsb[b]
            for s in range(C):
                def row_body(r, accs, _s=s):
                    base = _s * L + r
                    return tuple(
                        accs[jj] + rows_v[base, pl.ds(jj * 16, 16)] for jj in range(4)
                    )

                accs = lax.fori_loop(
                    0, L, row_body,
                    tuple(jnp.zeros((16,), jnp.float32) for _ in range(4)),
                    unroll=10,
                )
                for jj in range(4):
                    out_v[s, pl.ds(jj * 16, 16)] = accs[jj] * inv
            base_s = wid * SENT_PER_W + c * C
            pltpu.sync_copy(out_v, out_hbm.at[pl.ds(base_s, C)])
        return carry

    lax.fori_loop(0, NCHUNK // 2, outer, 0)


BM = 512
BN = 512


def _loss_body(s1_ref, s2_ref, out_ref):
    i = pl.program_id(0)
    j = pl.program_id(1)
    logits = lax.dot_general(
        s1_ref[...], s2_ref[...],
        dimension_numbers=(((1,), (1,)), ((), ())),
        preferred_element_type=jnp.float32,
    )
    loss = jnp.maximum(logits, 0.0) + jnp.log1p(jnp.exp(-jnp.abs(logits)))
    contrib = jnp.sum(loss)

    @pl.when((i == 0) & (j == 0))
    def _():
        out_ref[0, 0] = 0.0

    out_ref[0, 0] += contrib

    @pl.when(i == j)
    def _():
        row = lax.broadcasted_iota(jnp.int32, (BM, BN), 0)
        col = lax.broadcasted_iota(jnp.int32, (BM, BN), 1)
        diag = jnp.sum(jnp.where(row == col, logits, 0.0))
        out_ref[0, 0] += -diag


_loss_call = pl.pallas_call(
    _loss_body,
    grid=(B // BM, B // BN),
    in_specs=[
        pl.BlockSpec((BM, EMB), lambda i, j: (i, 0)),
        pl.BlockSpec((BN, EMB), lambda i, j: (j + B // BN, 0)),
    ],
    out_specs=pl.BlockSpec((1, 1), lambda i, j: (0, 0), memory_space=pltpu.SMEM),
    out_shape=jax.ShapeDtypeStruct((1, 1), jnp.float32),
)


def kernel(sentences1, sentences2, table):
    idx = jnp.concatenate([sentences1, sentences2], axis=0).reshape(-1)
    k = idx & (PW - 1)
    fidx = (idx & ~(PW - 1)) + jnp.where(k < HALF, k << 1, (k << 1) - (PW - 1))
    packed = _pack_call(table.T)
    table_lin = packed.reshape(NB * PW, EMB)
    pooled = _pooled_gather(fidx, table_lin)
    total = _loss_call(pooled, pooled)
    return total[0, 0] / jnp.float32(B * B)
